# baseline (speedup 1.0000x reference)
import jax
import jax.numpy as jnp
from jax.experimental import pallas as pl
from jax.experimental.pallas import tpu as pltpu
from jax.sharding import Mesh, PartitionSpec as P

B = 16
T = 12
N = 512
IN_DIM = 2
OUT_DIM = 1
HID = 64
N_SUP = 2
K_HOP = 2
N_PRED = 12
DX = 8
M = N_SUP * K_HOP + 1
NHOP = N_SUP * K_HOP


def _rowstack(v, nb):
    return jnp.concatenate(
        [v[:, b * N:(b + 1) * N] for b in range(nb)], axis=0)


def _unstack(big, d, nb):
    return [
        jnp.concatenate(
            [big[b * d:(b + 1) * d, m * N:(m + 1) * N] for b in range(nb)],
            axis=1)
        for m in range(NHOP)
    ]


def _cell(x, st, AM, ruWt, rub, cWt, cb, nb):
    dx = x.shape[0]
    bigx = jnp.dot(_rowstack(x, nb), AM, preferred_element_type=jnp.float32)
    xh = jnp.concatenate(_unstack(bigx, dx, nb), axis=0)

    def gconv(spart, Wt, bias):
        bigs = jnp.dot(_rowstack(spart, nb), AM,
                       preferred_element_type=jnp.float32)
        sh = jnp.concatenate(_unstack(bigs, HID, nb), axis=0)
        h = jnp.concatenate([x, spart, xh, sh], axis=0)
        return jnp.dot(Wt, h, preferred_element_type=jnp.float32) + bias

    ru = 1.0 / (1.0 + jnp.exp(-gconv(st, ruWt, rub)))
    r = ru[:HID]
    u = ru[HID:]
    c = jnp.tanh(gconv(r * st, cWt, cb))
    return u * st + (1.0 - u) * c


def _make_body(nb):
    bn = nb * N

    def _dcrnn_kernel(xin_ref, A0_ref, A1_ref,
                      e0ruW_ref, e0rub_ref, e0cW_ref, e0cb_ref,
                      e1ruW_ref, e1rub_ref, e1cW_ref, e1cb_ref,
                      d0ruW_ref, d0rub_ref, d0cW_ref, d0cb_ref,
                      d1ruW_ref, d1rub_ref, d1cW_ref, d1cb_ref,
                      doW_ref, dob_ref,
                      out_ref,
                      st0_ref, st1_ref, xd_ref, AM_ref):
        A0 = A0_ref[...]
        A1 = A1_ref[...]
        AM_ref[:, 0:N] = A0
        AM_ref[:, N:2 * N] = jnp.dot(A0, A0, preferred_element_type=jnp.float32)
        AM_ref[:, 2 * N:3 * N] = A1
        AM_ref[:, 3 * N:4 * N] = jnp.dot(A1, A1, preferred_element_type=jnp.float32)
        AM = AM_ref[...]
        e0 = (e0ruW_ref[...], e0rub_ref[...], e0cW_ref[...], e0cb_ref[...])
        e1 = (e1ruW_ref[...], e1rub_ref[...], e1cW_ref[...], e1cb_ref[...])
        d0 = (d0ruW_ref[...], d0rub_ref[...], d0cW_ref[...], d0cb_ref[...])
        d1 = (d1ruW_ref[...], d1rub_ref[...], d1cW_ref[...], d1cb_ref[...])

        st0_ref[...] = jnp.zeros((HID, bn), jnp.float32)
        st1_ref[...] = jnp.zeros((HID, bn), jnp.float32)

        def enc_body(t, carry):
            s0 = _cell(xin_ref[t], st0_ref[...], AM, *e0, nb)
            st0_ref[...] = s0
            s1 = _cell(s0, st1_ref[...], AM, *e1, nb)
            st1_ref[...] = s1
            return carry

        jax.lax.fori_loop(0, T, enc_body, 0)

        xd_ref[...] = jnp.zeros((DX, bn), jnp.float32)

        def dec_body(t, carry):
            s0 = _cell(xd_ref[...], st0_ref[...], AM, *d0, nb)
            st0_ref[...] = s0
            s1 = _cell(s0, st1_ref[...], AM, *d1, nb)
            st1_ref[...] = s1
            p = jnp.dot(doW_ref[...], s1,
                        preferred_element_type=jnp.float32) + dob_ref[...]
            out_ref[t] = p
            xd_ref[...] = p
            return carry

        jax.lax.fori_loop(0, N_PRED, dec_body, 0)

    return _dcrnn_kernel


def _forward(inputs, supports, weights):
    f32 = jnp.float32
    nb = inputs.shape[0]
    bn = nb * N

    xin = jnp.transpose(inputs, (1, 3, 0, 2)).reshape(T, IN_DIM, bn)
    xin = jnp.concatenate(
        [xin, jnp.zeros((T, DX - IN_DIM, bn), f32)], axis=1).astype(f32)

    A0 = jnp.transpose(supports[0]).astype(f32)
    A1 = jnp.transpose(supports[1]).astype(f32)

    out = pl.pallas_call(
        _make_body(nb),
        out_shape=jax.ShapeDtypeStruct((N_PRED, DX, bn), f32),
        scratch_shapes=[
            pltpu.VMEM((HID, bn), f32),
            pltpu.VMEM((HID, bn), f32),
            pltpu.VMEM((DX, bn), f32),
            pltpu.VMEM((N, NHOP * N), f32),
        ],
    )(xin, A0, A1, *weights)

    preds = out[:, 0, :].reshape(N_PRED, nb, N)
    return jnp.transpose(preds, (1, 0, 2))[..., None]


def kernel(inputs, supports, batch_seen,
           enc0_ru_W, enc0_ru_b, enc0_c_W, enc0_c_b,
           enc1_ru_W, enc1_ru_b, enc1_c_W, enc1_c_b,
           dec0_ru_W, dec0_ru_b, dec0_c_W, dec0_c_b,
           dec1_ru_W, dec1_ru_b, dec1_c_W, dec1_c_b,
           dec_out_W, dec_out_b):
    f32 = jnp.float32

    def prep(W, b, dx, dxp):
        din = dx + HID
        dout = W.shape[1]
        Wt = jnp.transpose(W)
        xpad = jnp.zeros((dout, dxp - dx), f32)
        xcols = []
        scols = []
        for m in range(M):
            blk = Wt[:, m * din:(m + 1) * din]
            xcols.append(jnp.concatenate([blk[:, :dx], xpad], axis=1))
            scols.append(blk[:, dx:])
        cols = [xcols[0], scols[0]] + xcols[1:] + scols[1:]
        return (jnp.concatenate(cols, axis=1).astype(f32),
                b.reshape(-1, 1).astype(f32))

    e0ruW, e0rub = prep(enc0_ru_W, enc0_ru_b, IN_DIM, DX)
    e0cW, e0cb = prep(enc0_c_W, enc0_c_b, IN_DIM, DX)
    e1ruW, e1rub = prep(enc1_ru_W, enc1_ru_b, HID, HID)
    e1cW, e1cb = prep(enc1_c_W, enc1_c_b, HID, HID)
    d0ruW, d0rub = prep(dec0_ru_W, dec0_ru_b, OUT_DIM, DX)
    d0cW, d0cb = prep(dec0_c_W, dec0_c_b, OUT_DIM, DX)
    d1ruW, d1rub = prep(dec1_ru_W, dec1_ru_b, HID, HID)
    d1cW, d1cb = prep(dec1_c_W, dec1_c_b, HID, HID)

    doW = jnp.concatenate(
        [jnp.transpose(dec_out_W), jnp.zeros((DX - OUT_DIM, HID), f32)], axis=0)
    dob = jnp.concatenate(
        [dec_out_b.reshape(OUT_DIM, 1), jnp.zeros((DX - OUT_DIM, 1), f32)], axis=0)

    weights = (e0ruW, e0rub, e0cW, e0cb,
               e1ruW, e1rub, e1cW, e1cb,
               d0ruW, d0rub, d0cW, d0cb,
               d1ruW, d1rub, d1cW, d1cb,
               doW, dob)

    inputs = inputs.astype(f32)
    supports = supports.astype(f32)

    devs = jax.devices()
    nd = len(devs)
    while nd > 1 and (B % nd != 0 or devs[0].platform != "tpu"):
        nd -= 1
    if nd > 1:
        mesh = Mesh(devs[:nd], ("b",))
        fwd = jax.shard_map(
            lambda i, s, w: _forward(i, s, w),
            mesh=mesh,
            in_specs=(P("b"), P(), P()),
            out_specs=P("b"),
            check_vma=False,
        )
        return fwd(inputs, supports, weights)
    return _forward(inputs, supports, weights)


# speedup vs baseline: 2.0305x; 2.0305x over previous
import jax
import jax.numpy as jnp
from jax.experimental import pallas as pl
from jax.experimental.pallas import tpu as pltpu

B = 16
T = 12
N = 512
IN_DIM = 2
OUT_DIM = 1
HID = 64
N_SUP = 2
K_HOP = 2
N_PRED = 12
DX = 8
M = N_SUP * K_HOP + 1
NHOP = N_SUP * K_HOP


def _rowstack(v, nb):
    return jnp.concatenate(
        [v[:, b * N:(b + 1) * N] for b in range(nb)], axis=0)


def _unstack(big, d, nb):
    return [
        jnp.concatenate(
            [big[b * d:(b + 1) * d, m * N:(m + 1) * N] for b in range(nb)],
            axis=1)
        for m in range(NHOP)
    ]


def _cell(x, st, AM, ruWt, rub, cWt, cb, nb):
    dx = x.shape[0]
    bigx = jnp.dot(_rowstack(x, nb), AM, preferred_element_type=jnp.float32)
    xh = jnp.concatenate(_unstack(bigx, dx, nb), axis=0)

    def gconv(spart, Wt, bias):
        bigs = jnp.dot(_rowstack(spart, nb), AM,
                       preferred_element_type=jnp.float32)
        sh = jnp.concatenate(_unstack(bigs, HID, nb), axis=0)
        h = jnp.concatenate([x, spart, xh, sh], axis=0)
        return jnp.dot(Wt, h, preferred_element_type=jnp.float32) + bias

    ru = 1.0 / (1.0 + jnp.exp(-gconv(st, ruWt, rub)))
    r = ru[:HID]
    u = ru[HID:]
    c = jnp.tanh(gconv(r * st, cWt, cb))
    return u * st + (1.0 - u) * c


def _make_body(nb):
    bn = nb * N

    def _dcrnn_kernel(xin_ref, A0_ref, A1_ref,
                      e0ruW_ref, e0rub_ref, e0cW_ref, e0cb_ref,
                      e1ruW_ref, e1rub_ref, e1cW_ref, e1cb_ref,
                      d0ruW_ref, d0rub_ref, d0cW_ref, d0cb_ref,
                      d1ruW_ref, d1rub_ref, d1cW_ref, d1cb_ref,
                      doW_ref, dob_ref,
                      out_ref,
                      st0_ref, st1_ref, xd_ref, AM_ref):
        A0 = A0_ref[...]
        A1 = A1_ref[...]
        AM_ref[:, 0:N] = A0
        AM_ref[:, N:2 * N] = jnp.dot(A0, A0, preferred_element_type=jnp.float32)
        AM_ref[:, 2 * N:3 * N] = A1
        AM_ref[:, 3 * N:4 * N] = jnp.dot(A1, A1, preferred_element_type=jnp.float32)
        AM = AM_ref[...]
        e0 = (e0ruW_ref[...], e0rub_ref[...], e0cW_ref[...], e0cb_ref[...])
        e1 = (e1ruW_ref[...], e1rub_ref[...], e1cW_ref[...], e1cb_ref[...])
        d0 = (d0ruW_ref[...], d0rub_ref[...], d0cW_ref[...], d0cb_ref[...])
        d1 = (d1ruW_ref[...], d1rub_ref[...], d1cW_ref[...], d1cb_ref[...])

        st0_ref[...] = jnp.zeros((HID, bn), jnp.float32)
        st1_ref[...] = jnp.zeros((HID, bn), jnp.float32)

        def enc_body(t, carry):
            s0 = _cell(xin_ref[t], st0_ref[...], AM, *e0, nb)
            st0_ref[...] = s0
            s1 = _cell(s0, st1_ref[...], AM, *e1, nb)
            st1_ref[...] = s1
            return carry

        jax.lax.fori_loop(0, T, enc_body, 0)

        xd_ref[...] = jnp.zeros((DX, bn), jnp.float32)

        def dec_body(t, carry):
            s0 = _cell(xd_ref[...], st0_ref[...], AM, *d0, nb)
            st0_ref[...] = s0
            s1 = _cell(s0, st1_ref[...], AM, *d1, nb)
            st1_ref[...] = s1
            p = jnp.dot(doW_ref[...], s1,
                        preferred_element_type=jnp.float32) + dob_ref[...]
            out_ref[t] = p
            xd_ref[...] = p
            return carry

        jax.lax.fori_loop(0, N_PRED, dec_body, 0)

    return _dcrnn_kernel


def _forward(inputs, supports, weights):
    f32 = jnp.float32
    nb = inputs.shape[0]
    bn = nb * N

    xin = jnp.transpose(inputs, (1, 3, 0, 2)).reshape(T, IN_DIM, bn)
    xin = jnp.concatenate(
        [xin, jnp.zeros((T, DX - IN_DIM, bn), f32)], axis=1).astype(f32)

    A0 = jnp.transpose(supports[0]).astype(f32)
    A1 = jnp.transpose(supports[1]).astype(f32)

    out = pl.pallas_call(
        _make_body(nb),
        out_shape=jax.ShapeDtypeStruct((N_PRED, DX, bn), f32),
        scratch_shapes=[
            pltpu.VMEM((HID, bn), f32),
            pltpu.VMEM((HID, bn), f32),
            pltpu.VMEM((DX, bn), f32),
            pltpu.VMEM((N, NHOP * N), f32),
        ],
    )(xin, A0, A1, *weights)

    preds = out[:, 0, :].reshape(N_PRED, nb, N)
    return jnp.transpose(preds, (1, 0, 2))[..., None]


def kernel(inputs, supports, batch_seen,
           enc0_ru_W, enc0_ru_b, enc0_c_W, enc0_c_b,
           enc1_ru_W, enc1_ru_b, enc1_c_W, enc1_c_b,
           dec0_ru_W, dec0_ru_b, dec0_c_W, dec0_c_b,
           dec1_ru_W, dec1_ru_b, dec1_c_W, dec1_c_b,
           dec_out_W, dec_out_b):
    f32 = jnp.float32

    def prep(W, b, dx, dxp):
        din = dx + HID
        dout = W.shape[1]
        Wt = jnp.transpose(W)
        xpad = jnp.zeros((dout, dxp - dx), f32)
        xcols = []
        scols = []
        for m in range(M):
            blk = Wt[:, m * din:(m + 1) * din]
            xcols.append(jnp.concatenate([blk[:, :dx], xpad], axis=1))
            scols.append(blk[:, dx:])
        cols = [xcols[0], scols[0]] + xcols[1:] + scols[1:]
        return (jnp.concatenate(cols, axis=1).astype(f32),
                b.reshape(-1, 1).astype(f32))

    e0ruW, e0rub = prep(enc0_ru_W, enc0_ru_b, IN_DIM, DX)
    e0cW, e0cb = prep(enc0_c_W, enc0_c_b, IN_DIM, DX)
    e1ruW, e1rub = prep(enc1_ru_W, enc1_ru_b, HID, HID)
    e1cW, e1cb = prep(enc1_c_W, enc1_c_b, HID, HID)
    d0ruW, d0rub = prep(dec0_ru_W, dec0_ru_b, OUT_DIM, DX)
    d0cW, d0cb = prep(dec0_c_W, dec0_c_b, OUT_DIM, DX)
    d1ruW, d1rub = prep(dec1_ru_W, dec1_ru_b, HID, HID)
    d1cW, d1cb = prep(dec1_c_W, dec1_c_b, HID, HID)

    doW = jnp.concatenate(
        [jnp.transpose(dec_out_W), jnp.zeros((DX - OUT_DIM, HID), f32)], axis=0)
    dob = jnp.concatenate(
        [dec_out_b.reshape(OUT_DIM, 1), jnp.zeros((DX - OUT_DIM, 1), f32)], axis=0)

    weights = (e0ruW, e0rub, e0cW, e0cb,
               e1ruW, e1rub, e1cW, e1cb,
               d0ruW, d0rub, d0cW, d0cb,
               d1ruW, d1rub, d1cW, d1cb,
               doW, dob)

    return _forward(inputs.astype(f32), supports.astype(f32), weights)
